# baseline (speedup 1.0000x reference)
import functools

import jax
import jax.numpy as jnp
from jax import lax
from jax.experimental import pallas as pl
from jax.experimental.pallas import tpu as pltpu
from jax.experimental.pallas import tpu_sc as plsc

N = 10000
E = 320000
D = 128
NC = 2
NS = 16
NW = NC * NS
L = 16

C = 80
KB = 8
EP = 327680
NCH = EP // C
EPW_A = EP // NW
NB_A = EPW_A // (KB * C)
EPW_B = EP // NS
NB_B = EPW_B // (KB * C)
EPW_H = E // NW
NP = 10240
RPT = NP // NS

_MESH = functools.partial(
    plsc.VectorSubcoreMesh, core_axis_name="c", subcore_axis_name="s")


def _hist_body(ei_hbm, zn_hbm, out_hbm, e0_v, e1_v, h0_v, h1_v):
    c = lax.axis_index("c")
    s = lax.axis_index("s")
    wid = c * NS + s
    base = wid * EPW_H
    pltpu.sync_copy(ei_hbm.at[pl.ds(base, EPW_H)], e0_v)
    pltpu.sync_copy(ei_hbm.at[pl.ds(E + base, EPW_H)], e1_v)
    pltpu.sync_copy(zn_hbm, h0_v)
    pltpu.sync_copy(zn_hbm, h1_v)
    ones16 = jnp.ones((L,), jnp.float32)
    UNR = 5

    def hloop(i, carry):
        for k in range(UNR):
            i0 = e0_v[pl.ds(i * (L * UNR) + k * L, L)]
            i1 = e1_v[pl.ds(i * (L * UNR) + k * L, L)]
            plsc.addupdate_scatter(h0_v, [i0], ones16)
            plsc.addupdate_scatter(h1_v, [i1], ones16)
        return carry

    lax.fori_loop(0, EPW_H // (L * UNR), hloop, 0)
    pltpu.sync_copy(h0_v, out_hbm.at[wid, 0])
    pltpu.sync_copy(h1_v, out_hbm.at[wid, 1])


def _hist_call(edge_index, zn):
    f = pl.kernel(
        _hist_body,
        out_type=jax.ShapeDtypeStruct((NW, 2, N), jnp.float32),
        mesh=_MESH(),
        compiler_params=pltpu.CompilerParams(needs_layout_passes=False),
        scratch_types=[
            pltpu.VMEM((EPW_H,), jnp.int32),
            pltpu.VMEM((EPW_H,), jnp.int32),
            pltpu.VMEM((N,), jnp.float32),
            pltpu.VMEM((N,), jnp.float32),
        ],
    )
    return f(edge_index, zn)


def _make_edge_acc(nb, gdim, sdim, split_cores, core_offset):
    def body(table_hbm, eip_hbm, znd_hbm, out_hbm,
             iin_v, iout_v, rows_v, acc_sh, gsem0, gsem1, isem):
        c = lax.axis_index("c")
        s = lax.axis_index("s")
        pltpu.sync_copy(znd_hbm.at[pl.ds(s * RPT, RPT)],
                        acc_sh.at[pl.ds(s * RPT, RPT)])
        plsc.subcore_barrier()

        if split_cores:
            crow = (c * NS + s) * (nb * KB)
        else:
            crow = s * (nb * KB)

        def load_idx(rowbase, slot):
            pltpu.async_copy(eip_hbm.at[gdim, pl.ds(rowbase, KB)],
                             iin_v.at[slot], isem)
            pltpu.async_copy(eip_hbm.at[sdim, pl.ds(rowbase, KB)],
                             iout_v.at[slot], isem)

        def wait_idx(rowbase, slot):
            pltpu.make_async_copy(eip_hbm.at[gdim, pl.ds(rowbase, KB)],
                                  iin_v.at[slot], isem).wait()
            pltpu.make_async_copy(eip_hbm.at[sdim, pl.ds(rowbase, KB)],
                                  iout_v.at[slot], isem).wait()

        def add_core_offset(slot):
            if core_offset:
                off = jnp.broadcast_to((c * core_offset).astype(jnp.int32),
                                       (L,))
                for r in range(KB):
                    for k in range(C // L):
                        iin_v[slot, r, pl.ds(k * L, L)] = (
                            iin_v[slot, r, pl.ds(k * L, L)] + off)

        load_idx(crow, 0)
        wait_idx(crow, 0)
        add_core_offset(0)

        def bloop(bi, carry):
            pb = lax.rem(bi, 2)

            @pl.when(bi + 1 < nb)
            def _prefetch_idx():
                load_idx(crow + (bi + 1) * KB, 1 - pb)

            gsems = (gsem0, gsem1)
            d = pltpu.async_copy(
                table_hbm.at[iin_v.at[pb, 0]], rows_v.at[0], gsems[0])
            for j in range(KB):
                p = j % 2
                if j + 1 < KB:
                    dn = pltpu.async_copy(
                        table_hbm.at[iin_v.at[pb, j + 1]],
                        rows_v.at[1 - p], gsems[(j + 1) % 2])
                d.wait()
                pltpu.sync_copy(rows_v.at[p],
                                acc_sh.at[iout_v.at[pb, j]], add=True)
                if j + 1 < KB:
                    d = dn

            @pl.when(bi + 1 < nb)
            def _wait_idx():
                wait_idx(crow + (bi + 1) * KB, 1 - pb)

            add_core_offset(1 - pb)
            return carry

        lax.fori_loop(0, nb, bloop, 0)
        plsc.subcore_barrier()
        pltpu.sync_copy(acc_sh.at[pl.ds(s * RPT, RPT)],
                        out_hbm.at[c, pl.ds(s * RPT, RPT)])

    def call(table, eip, znd):
        f = pl.kernel(
            body,
            out_type=jax.ShapeDtypeStruct((NC, NP, D), jnp.float32),
            mesh=_MESH(),
            scratch_types=[
                pltpu.VMEM((2, KB, C), jnp.int32),
                pltpu.VMEM((2, KB, C), jnp.int32),
                pltpu.VMEM((2, C, D), jnp.float32),
                pltpu.VMEM_SHARED((NP, D), jnp.float32),
                pltpu.SemaphoreType.DMA,
                pltpu.SemaphoreType.DMA,
                pltpu.SemaphoreType.DMA,
            ],
        )
        return f(table, eip, znd)

    return call


_edge_acc_a = _make_edge_acc(NB_A, 0, 1, True, 0)
_edge_acc_b = _make_edge_acc(NB_B, 1, 0, False, N)


def _dinv_of(h):
    return lax.rsqrt(1.0 + jnp.sum(h[:, NW:], axis=1, keepdims=True))


def _prep_tc(hist_t_ref, x_ref, xd_ref):
    xd_ref[...] = x_ref[...] * _dinv_of(hist_t_ref[...])


def _prep_call(hist_t, x):
    return pl.pallas_call(
        _prep_tc,
        out_shape=jax.ShapeDtypeStruct((N, D), jnp.float32),
    )(hist_t, x)


def _mid_tc(acc_ref, x_ref, hist_t_ref, w_ref, b_ref, t2_ref):
    dinv = _dinv_of(hist_t_ref[...])
    m = ((acc_ref[0, :N] + acc_ref[1, :N]) * dinv
         + x_ref[...] * (dinv * dinv))
    h = jnp.dot(m, w_ref[...], preferred_element_type=jnp.float32)
    h = jnp.maximum(h + b_ref[...], 0.0)
    t2_ref[0] = h
    t2_ref[1] = h * h


def _mid_call(acc, x, hist_t, w, b2):
    return pl.pallas_call(
        _mid_tc,
        out_shape=jax.ShapeDtypeStruct((2, N, D), jnp.float32),
    )(acc, x, hist_t, w, b2)


def _final_tc(sb_ref, t2_ref, hist_t_ref, gg_ref):
    h = t2_ref[0]
    cnt = jnp.sum(hist_t_ref[...][:, :NW], axis=1, keepdims=True)
    ssum = cnt * h * h - 2.0 * h * sb_ref[0, :N] + sb_ref[1, :N]
    ssum = jnp.maximum(ssum, 0.0)
    gg_ref[...] = jnp.tanh(ssum / jnp.maximum(cnt, 1.0))


def _final_call(sb, t2, hist_t):
    return pl.pallas_call(
        _final_tc,
        out_shape=jax.ShapeDtypeStruct((N, D), jnp.float32),
    )(sb, t2, hist_t)


def kernel(X, edge_index, W, b):
    zn = jnp.zeros((N,), jnp.float32)
    znd = jnp.zeros((NP, D), jnp.float32)

    hist = _hist_call(edge_index.reshape(2 * E), zn)
    hist_t = jnp.transpose(hist, (2, 1, 0)).reshape(N, 2 * NW)
    xd = _prep_call(hist_t, X)

    pad_idx = jnp.arange(EP - E, dtype=jnp.int32)
    pad_lo = pad_idx % N
    pad_hi = N + pad_idx % (NP - N)
    eip_a = jnp.concatenate(
        [edge_index, jnp.stack([pad_lo, pad_hi])], axis=1).reshape(2, NCH, C)
    eip_b = jnp.concatenate(
        [edge_index, jnp.stack([pad_hi, pad_lo])], axis=1).reshape(2, NCH, C)

    acc = _edge_acc_a(xd, eip_a, znd)
    t2 = _mid_call(acc, X, hist_t, W, b.reshape(1, D))
    sb = _edge_acc_b(t2.reshape(2 * N, D), eip_b, znd)
    return _final_call(sb, t2, hist_t)


# speedup vs baseline: 1.0684x; 1.0684x over previous
import functools

import jax
import jax.numpy as jnp
from jax import lax
from jax.experimental import pallas as pl
from jax.experimental.pallas import tpu as pltpu
from jax.experimental.pallas import tpu_sc as plsc

N = 10000
E = 320000
D = 128
NC = 2
NS = 16
NW = NC * NS
L = 16

C = 80
KB = 32
EP = 327680
NCH = EP // C
EPW_A = EP // NW
NB_A = EPW_A // (KB * C)
EPW_B = EP // NS
NB_B = EPW_B // (KB * C)
EPW_H = E // NW
NP = 10240
RPT = NP // NS

_MESH = functools.partial(
    plsc.VectorSubcoreMesh, core_axis_name="c", subcore_axis_name="s")


def _hist_body(ei_hbm, zn_hbm, out_hbm, e0_v, e1_v, h0_v, h1_v):
    c = lax.axis_index("c")
    s = lax.axis_index("s")
    wid = c * NS + s
    base = wid * EPW_H
    pltpu.sync_copy(ei_hbm.at[pl.ds(base, EPW_H)], e0_v)
    pltpu.sync_copy(ei_hbm.at[pl.ds(E + base, EPW_H)], e1_v)
    pltpu.sync_copy(zn_hbm, h0_v)
    pltpu.sync_copy(zn_hbm, h1_v)
    ones16 = jnp.ones((L,), jnp.float32)
    UNR = 5

    def hloop(i, carry):
        for k in range(UNR):
            i0 = e0_v[pl.ds(i * (L * UNR) + k * L, L)]
            i1 = e1_v[pl.ds(i * (L * UNR) + k * L, L)]
            plsc.addupdate_scatter(h0_v, [i0], ones16)
            plsc.addupdate_scatter(h1_v, [i1], ones16)
        return carry

    lax.fori_loop(0, EPW_H // (L * UNR), hloop, 0)
    pltpu.sync_copy(h0_v, out_hbm.at[wid, 0])
    pltpu.sync_copy(h1_v, out_hbm.at[wid, 1])


def _hist_call(edge_index, zn):
    f = pl.kernel(
        _hist_body,
        out_type=jax.ShapeDtypeStruct((NW, 2, N), jnp.float32),
        mesh=_MESH(),
        compiler_params=pltpu.CompilerParams(needs_layout_passes=False),
        scratch_types=[
            pltpu.VMEM((EPW_H,), jnp.int32),
            pltpu.VMEM((EPW_H,), jnp.int32),
            pltpu.VMEM((N,), jnp.float32),
            pltpu.VMEM((N,), jnp.float32),
        ],
    )
    return f(edge_index, zn)


def _make_edge_acc(nb, gdim, sdim, split_cores, core_offset):
    def body(table_hbm, eip_hbm, znd_hbm, out_hbm,
             iin_v, iout_v, rows_v, acc_sh, gsem0, gsem1, isem):
        c = lax.axis_index("c")
        s = lax.axis_index("s")
        pltpu.sync_copy(znd_hbm.at[pl.ds(s * RPT, RPT)],
                        acc_sh.at[pl.ds(s * RPT, RPT)])
        plsc.subcore_barrier()

        if split_cores:
            crow = (c * NS + s) * (nb * KB)
        else:
            crow = s * (nb * KB)

        def load_idx(rowbase, slot):
            pltpu.async_copy(eip_hbm.at[gdim, pl.ds(rowbase, KB)],
                             iin_v.at[slot], isem)
            pltpu.async_copy(eip_hbm.at[sdim, pl.ds(rowbase, KB)],
                             iout_v.at[slot], isem)

        def wait_idx(rowbase, slot):
            pltpu.make_async_copy(eip_hbm.at[gdim, pl.ds(rowbase, KB)],
                                  iin_v.at[slot], isem).wait()
            pltpu.make_async_copy(eip_hbm.at[sdim, pl.ds(rowbase, KB)],
                                  iout_v.at[slot], isem).wait()

        def add_core_offset(slot):
            if core_offset:
                off = jnp.broadcast_to((c * core_offset).astype(jnp.int32),
                                       (L,))
                for r in range(KB):
                    for k in range(C // L):
                        iin_v[slot, r, pl.ds(k * L, L)] = (
                            iin_v[slot, r, pl.ds(k * L, L)] + off)

        load_idx(crow, 0)
        wait_idx(crow, 0)
        add_core_offset(0)

        def bloop(bi, carry):
            pb = lax.rem(bi, 2)

            @pl.when(bi + 1 < nb)
            def _prefetch_idx():
                load_idx(crow + (bi + 1) * KB, 1 - pb)

            gsems = (gsem0, gsem1)
            d = pltpu.async_copy(
                table_hbm.at[iin_v.at[pb, 0]], rows_v.at[0], gsems[0])
            for j in range(KB):
                p = j % 2
                if j + 1 < KB:
                    dn = pltpu.async_copy(
                        table_hbm.at[iin_v.at[pb, j + 1]],
                        rows_v.at[1 - p], gsems[(j + 1) % 2])
                d.wait()
                pltpu.sync_copy(rows_v.at[p],
                                acc_sh.at[iout_v.at[pb, j]], add=True)
                if j + 1 < KB:
                    d = dn

            @pl.when(bi + 1 < nb)
            def _wait_idx():
                wait_idx(crow + (bi + 1) * KB, 1 - pb)

            add_core_offset(1 - pb)
            return carry

        lax.fori_loop(0, nb, bloop, 0)
        plsc.subcore_barrier()
        pltpu.sync_copy(acc_sh.at[pl.ds(s * RPT, RPT)],
                        out_hbm.at[c, pl.ds(s * RPT, RPT)])

    def call(table, eip, znd):
        f = pl.kernel(
            body,
            out_type=jax.ShapeDtypeStruct((NC, NP, D), jnp.float32),
            mesh=_MESH(),
            scratch_types=[
                pltpu.VMEM((2, KB, C), jnp.int32),
                pltpu.VMEM((2, KB, C), jnp.int32),
                pltpu.VMEM((2, C, D), jnp.float32),
                pltpu.VMEM_SHARED((NP, D), jnp.float32),
                pltpu.SemaphoreType.DMA,
                pltpu.SemaphoreType.DMA,
                pltpu.SemaphoreType.DMA,
            ],
        )
        return f(table, eip, znd)

    return call


_edge_acc_a = _make_edge_acc(NB_A, 0, 1, True, 0)
_edge_acc_b = _make_edge_acc(NB_B, 1, 0, False, N)


def _dinv_of(h):
    return lax.rsqrt(1.0 + jnp.sum(h[:, NW:], axis=1, keepdims=True))


def _prep_tc(hist_t_ref, x_ref, xd_ref):
    xd_ref[...] = x_ref[...] * _dinv_of(hist_t_ref[...])


def _prep_call(hist_t, x):
    return pl.pallas_call(
        _prep_tc,
        out_shape=jax.ShapeDtypeStruct((N, D), jnp.float32),
    )(hist_t, x)


def _mid_tc(acc_ref, x_ref, hist_t_ref, w_ref, b_ref, t2_ref):
    dinv = _dinv_of(hist_t_ref[...])
    m = ((acc_ref[0, :N] + acc_ref[1, :N]) * dinv
         + x_ref[...] * (dinv * dinv))
    h = jnp.dot(m, w_ref[...], preferred_element_type=jnp.float32)
    h = jnp.maximum(h + b_ref[...], 0.0)
    t2_ref[0] = h
    t2_ref[1] = h * h


def _mid_call(acc, x, hist_t, w, b2):
    return pl.pallas_call(
        _mid_tc,
        out_shape=jax.ShapeDtypeStruct((2, N, D), jnp.float32),
    )(acc, x, hist_t, w, b2)


def _final_tc(sb_ref, t2_ref, hist_t_ref, gg_ref):
    h = t2_ref[0]
    cnt = jnp.sum(hist_t_ref[...][:, :NW], axis=1, keepdims=True)
    ssum = cnt * h * h - 2.0 * h * sb_ref[0, :N] + sb_ref[1, :N]
    ssum = jnp.maximum(ssum, 0.0)
    gg_ref[...] = jnp.tanh(ssum / jnp.maximum(cnt, 1.0))


def _final_call(sb, t2, hist_t):
    return pl.pallas_call(
        _final_tc,
        out_shape=jax.ShapeDtypeStruct((N, D), jnp.float32),
    )(sb, t2, hist_t)


def kernel(X, edge_index, W, b):
    zn = jnp.zeros((N,), jnp.float32)
    znd = jnp.zeros((NP, D), jnp.float32)

    hist = _hist_call(edge_index.reshape(2 * E), zn)
    hist_t = jnp.transpose(hist, (2, 1, 0)).reshape(N, 2 * NW)
    xd = _prep_call(hist_t, X)

    pad_idx = jnp.arange(EP - E, dtype=jnp.int32)
    pad_lo = pad_idx % N
    pad_hi = N + pad_idx % (NP - N)
    eip_a = jnp.concatenate(
        [edge_index, jnp.stack([pad_lo, pad_hi])], axis=1).reshape(2, NCH, C)
    eip_b = jnp.concatenate(
        [edge_index, jnp.stack([pad_hi, pad_lo])], axis=1).reshape(2, NCH, C)

    acc = _edge_acc_a(xd, eip_a, znd)
    t2 = _mid_call(acc, X, hist_t, W, b.reshape(1, D))
    sb = _edge_acc_b(t2.reshape(2 * N, D), eip_b, znd)
    return _final_call(sb, t2, hist_t)


# speedup vs baseline: 1.2451x; 1.1654x over previous
import functools

import jax
import jax.numpy as jnp
from jax import lax
from jax.experimental import pallas as pl
from jax.experimental.pallas import tpu as pltpu
from jax.experimental.pallas import tpu_sc as plsc

N = 10000
E = 320000
D = 128
NC = 2
NS = 16
NW = NC * NS
L = 16

C = 80
KB = 32
EP = 327680
NCH = EP // C
EPW_A = EP // NW
NB_A = EPW_A // (KB * C)
EPW_B = EP // NS
NB_B = EPW_B // (KB * C)
EPW_H = E // NW
NP = 10240
RPT = NP // NS

_MESH = functools.partial(
    plsc.VectorSubcoreMesh, core_axis_name="c", subcore_axis_name="s")


def _hist_body(ei_hbm, zn_hbm, out_hbm, e0_v, e1_v, h0_v, h1_v):
    c = lax.axis_index("c")
    s = lax.axis_index("s")
    wid = c * NS + s
    base = wid * EPW_H
    pltpu.sync_copy(ei_hbm.at[pl.ds(base, EPW_H)], e0_v)
    pltpu.sync_copy(ei_hbm.at[pl.ds(E + base, EPW_H)], e1_v)
    pltpu.sync_copy(zn_hbm, h0_v)
    pltpu.sync_copy(zn_hbm, h1_v)
    ones16 = jnp.ones((L,), jnp.float32)
    UNR = 5

    def hloop(i, carry):
        for k in range(UNR):
            i0 = e0_v[pl.ds(i * (L * UNR) + k * L, L)]
            i1 = e1_v[pl.ds(i * (L * UNR) + k * L, L)]
            plsc.addupdate_scatter(h0_v, [i0], ones16)
            plsc.addupdate_scatter(h1_v, [i1], ones16)
        return carry

    lax.fori_loop(0, EPW_H // (L * UNR), hloop, 0)
    pltpu.sync_copy(h0_v, out_hbm.at[wid, 0])
    pltpu.sync_copy(h1_v, out_hbm.at[wid, 1])


def _hist_call(edge_index, zn):
    f = pl.kernel(
        _hist_body,
        out_type=jax.ShapeDtypeStruct((NW, 2, N), jnp.float32),
        mesh=_MESH(),
        compiler_params=pltpu.CompilerParams(needs_layout_passes=False),
        scratch_types=[
            pltpu.VMEM((EPW_H,), jnp.int32),
            pltpu.VMEM((EPW_H,), jnp.int32),
            pltpu.VMEM((N,), jnp.float32),
            pltpu.VMEM((N,), jnp.float32),
        ],
    )
    return f(edge_index, zn)


def _make_edge_acc(nb, gdim, sdim, split_cores, core_offset):
    def body(table_hbm, eip_hbm, znd_hbm, out_hbm,
             iin_v, iout_v, rows_v, acc_sh, gsem0, gsem1, gsem2, isem):
        c = lax.axis_index("c")
        s = lax.axis_index("s")
        pltpu.sync_copy(znd_hbm.at[pl.ds(s * RPT, RPT)],
                        acc_sh.at[pl.ds(s * RPT, RPT)])
        plsc.subcore_barrier()

        if split_cores:
            crow = (c * NS + s) * (nb * KB)
        else:
            crow = s * (nb * KB)

        def load_idx(rowbase, slot):
            pltpu.async_copy(eip_hbm.at[gdim, pl.ds(rowbase, KB)],
                             iin_v.at[slot], isem)
            pltpu.async_copy(eip_hbm.at[sdim, pl.ds(rowbase, KB)],
                             iout_v.at[slot], isem)

        def wait_idx(rowbase, slot):
            pltpu.make_async_copy(eip_hbm.at[gdim, pl.ds(rowbase, KB)],
                                  iin_v.at[slot], isem).wait()
            pltpu.make_async_copy(eip_hbm.at[sdim, pl.ds(rowbase, KB)],
                                  iout_v.at[slot], isem).wait()

        def add_core_offset(slot):
            if core_offset:
                off = jnp.broadcast_to((c * core_offset).astype(jnp.int32),
                                       (L,))
                for r in range(KB):
                    for k in range(C // L):
                        iin_v[slot, r, pl.ds(k * L, L)] = (
                            iin_v[slot, r, pl.ds(k * L, L)] + off)

        load_idx(crow, 0)
        wait_idx(crow, 0)
        add_core_offset(0)

        gsems = (gsem0, gsem1, gsem2)

        def gissue(j, pb):
            pltpu.async_copy(table_hbm.at[iin_v.at[pb, j]],
                             rows_v.at[j % 3], gsems[j % 3])

        def gwait(j, pb):
            pltpu.make_async_copy(table_hbm.at[iin_v.at[pb, j]],
                                  rows_v.at[j % 3], gsems[j % 3]).wait()

        def bloop(bi, carry):
            pb = lax.rem(bi, 2)

            @pl.when(bi + 1 < nb)
            def _prefetch_idx():
                load_idx(crow + (bi + 1) * KB, 1 - pb)

            gissue(0, pb)
            gissue(1, pb)
            for j in range(KB):
                if j + 2 < KB:
                    gissue(j + 2, pb)
                gwait(j, pb)
                pltpu.sync_copy(rows_v.at[j % 3],
                                acc_sh.at[iout_v.at[pb, j]], add=True)

            @pl.when(bi + 1 < nb)
            def _wait_idx():
                wait_idx(crow + (bi + 1) * KB, 1 - pb)

            add_core_offset(1 - pb)
            return carry

        lax.fori_loop(0, nb, bloop, 0)
        plsc.subcore_barrier()
        pltpu.sync_copy(acc_sh.at[pl.ds(s * RPT, RPT)],
                        out_hbm.at[c, pl.ds(s * RPT, RPT)])

    def call(table, eip, znd):
        f = pl.kernel(
            body,
            out_type=jax.ShapeDtypeStruct((NC, NP, D), jnp.float32),
            mesh=_MESH(),
            scratch_types=[
                pltpu.VMEM((2, KB, C), jnp.int32),
                pltpu.VMEM((2, KB, C), jnp.int32),
                pltpu.VMEM((3, C, D), jnp.float32),
                pltpu.VMEM_SHARED((NP, D), jnp.float32),
                pltpu.SemaphoreType.DMA,
                pltpu.SemaphoreType.DMA,
                pltpu.SemaphoreType.DMA,
                pltpu.SemaphoreType.DMA,
            ],
        )
        return f(table, eip, znd)

    return call


_edge_acc_a = _make_edge_acc(NB_A, 0, 1, True, 0)
_edge_acc_b = _make_edge_acc(NB_B, 1, 0, False, N)


def _dinv_of(h):
    return lax.rsqrt(1.0 + jnp.sum(h[:, NW:], axis=1, keepdims=True))


def _prep_tc(hist_t_ref, x_ref, xd_ref):
    xd_ref[...] = x_ref[...] * _dinv_of(hist_t_ref[...])


def _prep_call(hist_t, x):
    return pl.pallas_call(
        _prep_tc,
        out_shape=jax.ShapeDtypeStruct((N, D), jnp.float32),
    )(hist_t, x)


def _mid_tc(acc_ref, x_ref, hist_t_ref, w_ref, b_ref, t2_ref):
    dinv = _dinv_of(hist_t_ref[...])
    m = ((acc_ref[0, :N] + acc_ref[1, :N]) * dinv
         + x_ref[...] * (dinv * dinv))
    h = jnp.dot(m, w_ref[...], preferred_element_type=jnp.float32)
    h = jnp.maximum(h + b_ref[...], 0.0)
    t2_ref[0] = h
    t2_ref[1] = h * h


def _mid_call(acc, x, hist_t, w, b2):
    return pl.pallas_call(
        _mid_tc,
        out_shape=jax.ShapeDtypeStruct((2, N, D), jnp.float32),
    )(acc, x, hist_t, w, b2)


def _final_tc(sb_ref, t2_ref, hist_t_ref, gg_ref):
    h = t2_ref[0]
    cnt = jnp.sum(hist_t_ref[...][:, :NW], axis=1, keepdims=True)
    ssum = cnt * h * h - 2.0 * h * sb_ref[0, :N] + sb_ref[1, :N]
    ssum = jnp.maximum(ssum, 0.0)
    gg_ref[...] = jnp.tanh(ssum / jnp.maximum(cnt, 1.0))


def _final_call(sb, t2, hist_t):
    return pl.pallas_call(
        _final_tc,
        out_shape=jax.ShapeDtypeStruct((N, D), jnp.float32),
    )(sb, t2, hist_t)


def kernel(X, edge_index, W, b):
    zn = jnp.zeros((N,), jnp.float32)
    znd = jnp.zeros((NP, D), jnp.float32)

    hist = _hist_call(edge_index.reshape(2 * E), zn)
    hist_t = jnp.transpose(hist, (2, 1, 0)).reshape(N, 2 * NW)
    xd = _prep_call(hist_t, X)

    pad_idx = jnp.arange(EP - E, dtype=jnp.int32)
    pad_lo = pad_idx % N
    pad_hi = N + pad_idx % (NP - N)
    eip_a = jnp.concatenate(
        [edge_index, jnp.stack([pad_lo, pad_hi])], axis=1).reshape(2, NCH, C)
    eip_b = jnp.concatenate(
        [edge_index, jnp.stack([pad_hi, pad_lo])], axis=1).reshape(2, NCH, C)

    acc = _edge_acc_a(xd, eip_a, znd)
    t2 = _mid_call(acc, X, hist_t, W, b.reshape(1, D))
    sb = _edge_acc_b(t2.reshape(2 * N, D), eip_b, znd)
    return _final_call(sb, t2, hist_t)
